# baseline (speedup 1.0000x reference)
import jax
import jax.numpy as jnp
from jax import lax
from jax.experimental import pallas as pl
from jax.experimental.pallas import tpu as pltpu
from jax.experimental.pallas import tpu_sc as plsc

K = 20
N = 8192
NW = 32
RPW = N // NW
OUTW = 32
L = 16
NCH = N // L
G = 4
BIGI = 2**30
INF = float("inf")


def _iota():
    return lax.iota(jnp.int32, L)


def _rne_bf16(v):
    bits = lax.bitcast_convert_type(v, jnp.uint32)
    r = bits + jnp.uint32(0x7FFF) + ((bits >> jnp.uint32(16)) & jnp.uint32(1))
    r = r & jnp.uint32(0xFFFF0000)
    return lax.bitcast_convert_type(r, jnp.float32)


def _dyn_gather(vec, idx):
    dnums = lax.GatherDimensionNumbers(
        offset_dims=(), collapsed_slice_dims=(0,), start_index_map=(0,))
    return lax.gather(vec, idx[:, None], dnums, (1,),
                      mode=lax.GatherScatterMode.PROMISE_IN_BOUNDS)


def _vmin_all(v):
    for s in (8, 4, 2, 1):
        v = jnp.minimum(v, _dyn_gather(v, _iota() ^ s))
    return v


def _scalar(ref, idx):
    return plsc.load_gather(ref, [jnp.full((L,), idx, jnp.int32)])[0]


def _body(pxh, pyh, pzh, bh, outh,
          bv, xbv, ybv, zbv, sqv, srv, erv, dbuf, cmbuf, outv):
    wid = lax.axis_index("s") * 2 + lax.axis_index("c")

    pltpu.sync_copy(pxh, xbv.at[pl.ds(0, N)])
    pltpu.sync_copy(pyh, ybv.at[pl.ds(0, N)])
    pltpu.sync_copy(pzh, zbv.at[pl.ds(0, N)])
    pltpu.sync_copy(bh, bv.at[pl.ds(0, N)])

    lbs = []
    for b in range(1, 9):
        lo = jnp.int32(0)
        for s in (4096, 2048, 1024, 512, 256, 128, 64, 32, 16, 8, 4, 2, 1):
            cand = lo + s
            v = _scalar(bv, cand - 1)
            lo = jnp.where(v < b, cand, lo)
        lbs.append(lo)
    start_vec = jnp.zeros((L,), jnp.int32)
    end_vec = jnp.full((L,), N, jnp.int32)
    for b in range(1, 9):
        start_vec = jnp.where(_iota() == b, lbs[b - 1], start_vec)
    for b in range(0, 8):
        end_vec = jnp.where(_iota() == b, lbs[b], end_vec)

    def stage(j, _):
        off = pl.multiple_of(j * L, L)
        qx = xbv[pl.ds(off, L)]
        qy = ybv[pl.ds(off, L)]
        qz = zbv[pl.ds(off, L)]
        sqv[pl.ds(off, L)] = (qx * qx + qy * qy) + qz * qz
        xbv[pl.ds(off, L)] = _rne_bf16(qx)
        ybv[pl.ds(off, L)] = _rne_bf16(qy)
        zbv[pl.ds(off, L)] = _rne_bf16(qz)
        bc = bv[pl.ds(off, L)]
        srv[pl.ds(off, L)] = _dyn_gather(start_vec, bc)
        erv[pl.ds(off, L)] = _dyn_gather(end_vec, bc)
        return 0

    lax.fori_loop(0, NCH, stage, 0)

    r0 = wid * RPW

    def row_body(i, _):
        rows = []
        for q in range(G):
            r = r0 + i * G + q
            rows.append((_scalar(srv, r), _scalar(erv, r), _scalar(sqv, r),
                         _scalar(xbv, r), _scalar(ybv, r), _scalar(zbv, r)))
        sc0 = rows[0][0] >> 8
        sc1 = (rows[G - 1][1] + 255) >> 8

        def phase_a(s, carry):
            mcm = [carry[2 * q] for q in range(G)]
            pcm = [carry[2 * q + 1] for q in range(G)]
            msc = [jnp.full((L,), INF, jnp.float32) for _ in range(G)]
            base = pl.multiple_of(s * 256, 256)
            for u in range(16):
                off = base + u * L
                qx = xbv[pl.ds(off, L)]
                qy = ybv[pl.ds(off, L)]
                qz = zbv[pl.ds(off, L)]
                sqc = sqv[pl.ds(off, L)]
                ii = off + _iota()
                for q in range(G):
                    sr, er, rsq, rxb, ryb, rzb = rows[q]
                    dot = (rxb * qx + ryb * qy) + rzb * qz
                    d = (rsq + sqc) - 2.0 * dot
                    d = jnp.where((ii >= sr) & (ii < er), d, INF)
                    dbuf[pl.ds(off + q * N, L)] = d
                    msc[q] = jnp.minimum(msc[q], d)
            cpos = s * L + _iota()
            out = []
            for q in range(G):
                cmbuf[pl.ds(pl.multiple_of(s * L, L) + q * NCH, L)] = msc[q]
                upd = msc[q] < mcm[q]
                out.append(jnp.where(upd, msc[q], mcm[q]))
                out.append(jnp.where(upd, cpos, pcm[q]))
            return tuple(out)

        init = (jnp.full((L,), INF, jnp.float32),
                jnp.full((L,), BIGI, jnp.int32)) * G
        st = lax.fori_loop(sc0, sc1, phase_a, init)

        def extract(t, carry):
            out = []
            for q in range(G):
                mcm, pcm, a0, a1 = carry[q * 4:q * 4 + 4]
                g = _vmin_all(mcm)
                spos = _vmin_all(jnp.where(mcm == g, pcm, BIGI))
                lcm = spos & 15
                cidx = (spos >> 4) * 256 + _iota() * L + lcm
                dd = plsc.load_gather(dbuf, [cidx + q * N])
                gp = _vmin_all(jnp.where(dd == g, cidx, BIGI))
                a0 = jnp.where(_iota() == t, gp, a0)
                a1 = jnp.where(_iota() == (t - 16), gp, a1)
                dd = jnp.where(cidx == gp, INF, dd)
                plsc.store_scatter(dbuf, [cidx + q * N], dd)
                nc = _vmin_all(dd)
                plsc.store_scatter(cmbuf, [spos + q * NCH], nc,
                                   mask=_iota() == 0)

                rm = jnp.full((L,), INF, jnp.float32)
                rpos = jnp.full((L,), BIGI, jnp.int32)
                for gi in range(2):
                    scs = sc0 + gi * L + _iota()
                    ok = scs < sc1
                    idx = jnp.where(ok, scs * L + lcm, 0)
                    col = plsc.load_gather(cmbuf, [idx + q * NCH])
                    col = jnp.where(ok, col, INF)
                    upd = col < rm
                    rm = jnp.where(upd, col, rm)
                    rpos = jnp.where(upd, idx, rpos)
                newm = _vmin_all(rm)
                newp = _vmin_all(jnp.where(rm == newm, rpos, BIGI))
                lmask = _iota() == lcm
                out.append(jnp.where(lmask, newm, mcm))
                out.append(jnp.where(lmask, newp, pcm))
                out.append(a0)
                out.append(a1)
            return tuple(out)

        z = jnp.zeros((L,), jnp.int32)
        init_ex = tuple(v for q in range(G)
                        for v in (st[2 * q], st[2 * q + 1], z, z))
        fin = lax.fori_loop(0, K, extract, init_ex)

        for q in range(G):
            roff = pl.multiple_of((i * G + q) * OUTW, OUTW)
            outv[pl.ds(roff, L)] = fin[q * 4 + 2]
            outv[pl.ds(roff + L, L)] = fin[q * 4 + 3]
        return 0

    lax.fori_loop(0, RPW // G, row_body, 0)

    pltpu.sync_copy(outv, outh.at[pl.ds(wid * RPW * OUTW, RPW * OUTW)])


def _sc_call(px, py, pz, b32):
    mesh = plsc.VectorSubcoreMesh(core_axis_name="c", subcore_axis_name="s",
                                  num_cores=2, num_subcores=16)
    fn = pl.kernel(
        _body,
        out_type=jax.ShapeDtypeStruct((N * OUTW,), jnp.int32),
        mesh=mesh,
        compiler_params=pltpu.CompilerParams(needs_layout_passes=False),
        scratch_types=[
            pltpu.VMEM((N + L,), jnp.int32),
            pltpu.VMEM((N + L,), jnp.float32),
            pltpu.VMEM((N + L,), jnp.float32),
            pltpu.VMEM((N + L,), jnp.float32),
            pltpu.VMEM((N + L,), jnp.float32),
            pltpu.VMEM((N + L,), jnp.int32),
            pltpu.VMEM((N + L,), jnp.int32),
            pltpu.VMEM((G * N,), jnp.float32),
            pltpu.VMEM((G * NCH,), jnp.float32),
            pltpu.VMEM((RPW * OUTW,), jnp.int32),
        ],
    )
    return fn(px, py, pz, b32)


def kernel(x, batch):
    xf = x.astype(jnp.float32)
    px = xf[:, 0]
    py = xf[:, 1]
    pz = xf[:, 2]
    b32 = batch.astype(jnp.int32)
    out = _sc_call(px, py, pz, b32)
    return out.reshape(N, OUTW)[:, :K]
